# Initial kernel scaffold; baseline (speedup 1.0000x reference)
import jax
import jax.numpy as jnp
from jax.experimental import pallas as pl


def kernel(x, edge_index, batch, params):
    raise NotImplementedError("write your pallas kernel here")


# speedup vs baseline: 4.4088x; 4.4088x over previous
import functools
import jax
import jax.numpy as jnp
from jax import lax
from jax.experimental import pallas as pl
from jax.experimental.pallas import tpu as pltpu
from jax.experimental.pallas import tpu_sc as plsc

_N = 10000
_E = 320000
_H = 128
_OUT = 64
_G = 128
_L = 4

_NC = 2
_NS = 16
_NW = _NC * _NS
_EPT = _E // _NW
_W = 80
_NWIN = _EPT // _W
_CPT = 10
_RPT = _N // _CPT
_ZR = 200
_NZ = _RPT // _ZR


def _make_sc_segment_sum():
  mesh = plsc.VectorSubcoreMesh(core_axis_name="c", subcore_axis_name="s")

  @functools.partial(
      pl.kernel,
      out_type=jax.ShapeDtypeStruct((_NC, _N, _H), jnp.float32),
      mesh=mesh,
      scratch_types=[
          pltpu.VMEM((_W,), jnp.int32),
          pltpu.VMEM((_W,), jnp.int32),
          pltpu.VMEM((_W, _H), jnp.float32),
          pltpu.VMEM((_ZR, _H), jnp.float32),
          pltpu.VMEM_SHARED((_N, _H), jnp.float32),
          pltpu.SemaphoreType.DMA,
      ],
  )
  def k(h_hbm, src_hbm, dst_hbm, out_hbm, sidx, didx, rows, zbuf, acc, sem):
    core = lax.axis_index("c")
    sub = lax.axis_index("s")
    wid = sub * _NC + core

    @pl.when(sub < _CPT)
    def _():
      @pl.loop(0, _ZR)
      def _(r):
        for c in range(_H // 16):
          zbuf[r, pl.ds(c * 16, 16)] = jnp.zeros((16,), jnp.float32)

      for j in range(_NZ):
        pltpu.sync_copy(zbuf, acc.at[pl.ds(sub * _RPT + j * _ZR, _ZR)])
    plsc.subcore_barrier()

    @pl.loop(0, _NWIN)
    def _(w):
      base = wid * _EPT + w * _W
      pltpu.sync_copy(src_hbm.at[pl.ds(base, _W)], sidx)
      pltpu.sync_copy(dst_hbm.at[pl.ds(base, _W)], didx)
      pltpu.async_copy(h_hbm.at[sidx], rows, sem).wait()
      pltpu.sync_copy(rows, acc.at[didx], add=True)

    plsc.subcore_barrier()

    @pl.when(sub < _CPT)
    def _():
      for j in range(_NZ):
        r0 = sub * _RPT + j * _ZR
        pltpu.sync_copy(acc.at[pl.ds(r0, _ZR)],
                        out_hbm.at[core, pl.ds(r0, _ZR)])

  return k


_sc_segment_sum = _make_sc_segment_sum()


def _bn(m, g, be):
  mu = jnp.mean(m, axis=0, keepdims=True)
  var = jnp.mean((m - mu) ** 2, axis=0, keepdims=True)
  return g * (m - mu) / jnp.sqrt(var + 1e-5) + be


def _tc_layer_body(pool_input, h_ref, a_ref, b_ref, w1, bb1, g1, be1,
                   w2, bb2, g2, be2, wf, bf, wf0, bf0, y_ref,
                   hout_ref, yout_ref):
  hp = jax.lax.Precision.HIGHEST
  h = h_ref[...]
  z = h + a_ref[0] + a_ref[1]
  m = jnp.dot(z, w1[...], precision=hp) + bb1[...]
  m = jnp.maximum(_bn(m, g1[...], be1[...]), 0.0)
  m = jnp.dot(m, w2[...], precision=hp) + bb2[...]
  m = jnp.maximum(_bn(m, g2[...], be2[...]), 0.0)
  hout_ref[...] = m
  onehot = (lax.broadcasted_iota(jnp.int32, (_G, _N), 0) ==
            b_ref[...]).astype(jnp.float32)
  y = y_ref[...] + jnp.dot(jnp.dot(onehot, m, precision=hp), wf[...],
                           precision=hp) + bf[...]
  if pool_input:
    y = y + jnp.dot(jnp.dot(onehot, h, precision=hp), wf0[...],
                    precision=hp) + bf0[...]
  yout_ref[...] = y


def _tc_layer(pool_input, h, agg, batch2d, p, fc, fc0, y):
  body = functools.partial(_tc_layer_body, pool_input)
  return pl.pallas_call(
      body,
      out_shape=(jax.ShapeDtypeStruct((_N, _H), jnp.float32),
                 jax.ShapeDtypeStruct((_G, _OUT), jnp.float32)),
  )(h, agg, batch2d,
    p['W1'], p['b1'].reshape(1, -1), p['g1'].reshape(1, -1),
    p['be1'].reshape(1, -1),
    p['W2'], p['b2'].reshape(1, -1), p['g'].reshape(1, -1),
    p['be'].reshape(1, -1),
    fc['W'], fc['b'].reshape(1, -1),
    fc0['W'], fc0['b'].reshape(1, -1), y)


def kernel(x, edge_index, batch, params):
  src = edge_index[0]
  dst = edge_index[1]
  batch2d = batch.reshape(1, _N)
  y = jnp.zeros((_G, _OUT), jnp.float32)
  h = x
  for i in range(_L):
    agg = _sc_segment_sum(h, src, dst)
    h, y = _tc_layer(i == 0, h, agg, batch2d, params['conv%d' % i],
                     params['fcs'][i + 1], params['fcs'][0], y)
  return y


# speedup vs baseline: 9.6726x; 2.1939x over previous
import functools
import jax
import jax.numpy as jnp
from jax import lax
from jax.experimental import pallas as pl
from jax.experimental.pallas import tpu as pltpu
from jax.experimental.pallas import tpu_sc as plsc

_N = 10000
_E = 320000
_H = 128
_OUT = 64
_G = 128
_L = 4

_NC = 2
_NS = 16
_NW = _NC * _NS
_EPT = _E // _NW
_W = 80
_NWIN = _EPT // _W
_CPT = 10
_RPT = _N // _CPT


def _make_sc_segment_sum():
  mesh = plsc.VectorSubcoreMesh(core_axis_name="c", subcore_axis_name="s")

  @functools.partial(
      pl.kernel,
      out_type=jax.ShapeDtypeStruct((_NC, _N, _H), jnp.float32),
      mesh=mesh,
      scratch_types=[
          pltpu.VMEM((_EPT,), jnp.int32),
          pltpu.VMEM((_NWIN, _W), jnp.int32),
          pltpu.VMEM((2, _W, _H), jnp.float32),
          pltpu.VMEM_SHARED((_N, _H), jnp.float32),
          pltpu.SemaphoreType.DMA,
          pltpu.SemaphoreType.DMA,
          pltpu.SemaphoreType.DMA,
      ],
  )
  def k(h_hbm, src_hbm, dst_hbm, out_hbm, sidx, didx, rows, acc,
        isem, gsem0, gsem1):
    core = lax.axis_index("c")
    sub = lax.axis_index("s")
    wid = sub * _NC + core

    ic0 = pltpu.async_copy(src_hbm.at[pl.ds(wid * _EPT, _EPT)], sidx, isem)
    ic1 = pltpu.async_copy(dst_hbm.at[wid], didx, isem)

    @pl.when(sub < _CPT)
    def _():
      r0 = sub * _RPT
      pltpu.sync_copy(h_hbm.at[pl.ds(r0, _RPT)], acc.at[pl.ds(r0, _RPT)])
    ic0.wait()
    ic1.wait()
    plsc.subcore_barrier()

    gsems = (gsem0, gsem1)

    def g_issue(w, b):
      pltpu.async_copy(h_hbm.at[sidx.at[pl.ds(w * _W, _W)]], rows.at[b],
                       gsems[b])

    def g_wait(b):
      pltpu.make_async_copy(h_hbm.at[sidx.at[pl.ds(0, _W)]], rows.at[b],
                            gsems[b]).wait()

    g_issue(0, 0)

    @pl.loop(0, _NWIN - 1, step=2)
    def _(w):
      g_issue(w + 1, 1)
      g_wait(0)
      pltpu.sync_copy(rows.at[0], acc.at[didx.at[w]], add=True)

      @pl.when(w + 2 < _NWIN)
      def _():
        g_issue(w + 2, 0)
      g_wait(1)
      pltpu.sync_copy(rows.at[1], acc.at[didx.at[w + 1]], add=True)

    g_wait(0)
    pltpu.sync_copy(rows.at[0], acc.at[didx.at[_NWIN - 1]], add=True)

    plsc.subcore_barrier()

    @pl.when(sub < _CPT)
    def _():
      r0 = sub * _RPT
      pltpu.sync_copy(acc.at[pl.ds(r0, _RPT)],
                      out_hbm.at[core, pl.ds(r0, _RPT)])

  return k


_sc_segment_sum = _make_sc_segment_sum()


def _bn(m, g, be):
  mu = jnp.mean(m, axis=0, keepdims=True)
  var = jnp.mean((m - mu) ** 2, axis=0, keepdims=True)
  return g * (m - mu) / jnp.sqrt(var + 1e-5) + be


def _tc_layer_body(pool_input, h_ref, a_ref, b_ref, w1, bb1, g1, be1,
                   w2, bb2, g2, be2, wf, bf, wf0, bf0, y_ref,
                   hout_ref, yout_ref):
  hp = jax.lax.Precision.HIGHEST
  h = h_ref[...]
  z = a_ref[0] + a_ref[1] - h
  m = jnp.dot(z, w1[...], precision=hp) + bb1[...]
  m = jnp.maximum(_bn(m, g1[...], be1[...]), 0.0)
  m = jnp.dot(m, w2[...], precision=hp) + bb2[...]
  m = jnp.maximum(_bn(m, g2[...], be2[...]), 0.0)
  hout_ref[...] = m
  onehot = (lax.broadcasted_iota(jnp.int32, (_G, _N), 0) ==
            b_ref[...]).astype(jnp.float32)
  y = y_ref[...] + jnp.dot(jnp.dot(onehot, m, precision=hp), wf[...],
                           precision=hp) + bf[...]
  if pool_input:
    y = y + jnp.dot(jnp.dot(onehot, h, precision=hp), wf0[...],
                    precision=hp) + bf0[...]
  yout_ref[...] = y


def _tc_layer(pool_input, h, agg, batch2d, p, fc, fc0, y):
  body = functools.partial(_tc_layer_body, pool_input)
  return pl.pallas_call(
      body,
      out_shape=(jax.ShapeDtypeStruct((_N, _H), jnp.float32),
                 jax.ShapeDtypeStruct((_G, _OUT), jnp.float32)),
  )(h, agg, batch2d,
    p['W1'], p['b1'].reshape(1, -1), p['g1'].reshape(1, -1),
    p['be1'].reshape(1, -1),
    p['W2'], p['b2'].reshape(1, -1), p['g'].reshape(1, -1),
    p['be'].reshape(1, -1),
    fc['W'], fc['b'].reshape(1, -1),
    fc0['W'], fc0['b'].reshape(1, -1), y)


def kernel(x, edge_index, batch, params):
  src = edge_index[0]
  dst = edge_index[1].reshape(_NW, _NWIN, _W)
  batch2d = batch.reshape(1, _N)
  y = jnp.zeros((_G, _OUT), jnp.float32)
  h = x
  for i in range(_L):
    agg = _sc_segment_sum(h, src, dst)
    h, y = _tc_layer(i == 0, h, agg, batch2d, params['conv%d' % i],
                     params['fcs'][i + 1], params['fcs'][0], y)
  return y


# speedup vs baseline: 10.9645x; 1.1336x over previous
import functools
import jax
import jax.numpy as jnp
from jax import lax
from jax.experimental import pallas as pl
from jax.experimental.pallas import tpu as pltpu
from jax.experimental.pallas import tpu_sc as plsc

_N = 10000
_E = 320000
_H = 128
_OUT = 64
_G = 128
_L = 4

_NC = 2
_NS = 16
_NW = _NC * _NS
_EPT = _E // _NW
_W = 80
_NWIN = _EPT // _W
_CPT = 10
_RPT = _N // _CPT


def _make_sc_segment_sum():
  mesh = plsc.VectorSubcoreMesh(core_axis_name="c", subcore_axis_name="s")

  @functools.partial(
      pl.kernel,
      out_type=jax.ShapeDtypeStruct((_NC, _N, _H), jnp.float32),
      mesh=mesh,
      scratch_types=[
          pltpu.VMEM((3, _W), jnp.int32),
          pltpu.VMEM((_NWIN, _W), jnp.int32),
          pltpu.VMEM((3, _W, _H), jnp.float32),
          pltpu.VMEM_SHARED((_N, _H), jnp.float32),
          pltpu.SemaphoreType.DMA,
          pltpu.SemaphoreType.DMA,
          pltpu.SemaphoreType.DMA,
          pltpu.SemaphoreType.DMA,
          pltpu.SemaphoreType.DMA,
          pltpu.SemaphoreType.DMA,
          pltpu.SemaphoreType.DMA,
      ],
  )
  def k(h_hbm, src_hbm, dst_hbm, out_hbm, sidx, didx, rows, acc,
        isem, gsem0, gsem1, gsem2, ssem0, ssem1, ssem2):
    core = lax.axis_index("c")
    sub = lax.axis_index("s")
    wid = sub * _NC + core
    ebase = wid * _EPT

    ic1 = pltpu.async_copy(dst_hbm.at[wid], didx, isem)

    @pl.when(sub < _CPT)
    def _():
      r0 = sub * _RPT
      pltpu.sync_copy(h_hbm.at[pl.ds(r0, _RPT)], acc.at[pl.ds(r0, _RPT)])
    ic1.wait()
    plsc.subcore_barrier()

    gsems = (gsem0, gsem1, gsem2)
    ssems = (ssem0, ssem1, ssem2)

    def sl_issue(w, b):
      pltpu.async_copy(src_hbm.at[pl.ds(ebase + w * _W, _W)], sidx.at[b],
                       ssems[b])

    def sl_wait(b):
      pltpu.make_async_copy(src_hbm.at[pl.ds(0, _W)], sidx.at[b],
                            ssems[b]).wait()

    def g_issue(b):
      pltpu.async_copy(h_hbm.at[sidx.at[b]], rows.at[b], gsems[b])

    def g_wait(b):
      pltpu.make_async_copy(h_hbm.at[sidx.at[0]], rows.at[b],
                            gsems[b]).wait()

    for b in range(3):
      sl_issue(b, b)
    for b in range(3):
      sl_wait(b)
      g_issue(b)

    @pl.loop(0, _NWIN - 2, step=3)
    def _(w):
      for b in range(3):
        ww = w + b
        g_wait(b)

        @pl.when(ww + 3 < _NWIN)
        def _():
          sl_issue(ww + 3, b)
        pltpu.sync_copy(rows.at[b], acc.at[didx.at[ww]], add=True)

        @pl.when(ww + 3 < _NWIN)
        def _():
          sl_wait(b)
          g_issue(b)

    for ww, b in ((_NWIN - 2, 0), (_NWIN - 1, 1)):
      g_wait(b)
      pltpu.sync_copy(rows.at[b], acc.at[didx.at[ww]], add=True)

    plsc.subcore_barrier()

    @pl.when(sub < _CPT)
    def _():
      r0 = sub * _RPT
      pltpu.sync_copy(acc.at[pl.ds(r0, _RPT)],
                      out_hbm.at[core, pl.ds(r0, _RPT)])

  return k


_sc_segment_sum = _make_sc_segment_sum()


def _bn(m, g, be):
  mu = jnp.mean(m, axis=0, keepdims=True)
  var = jnp.mean((m - mu) ** 2, axis=0, keepdims=True)
  return g * (m - mu) / jnp.sqrt(var + 1e-5) + be


def _tc_layer_body(pool_input, h_ref, a_ref, b_ref, w1, bb1, g1, be1,
                   w2, bb2, g2, be2, wf, bf, wf0, bf0, y_ref,
                   hout_ref, yout_ref):
  hp = jax.lax.Precision.HIGHEST
  h = h_ref[...]
  z = a_ref[0] + a_ref[1] - h
  m = jnp.dot(z, w1[...], precision=hp) + bb1[...]
  m = jnp.maximum(_bn(m, g1[...], be1[...]), 0.0)
  m = jnp.dot(m, w2[...], precision=hp) + bb2[...]
  m = jnp.maximum(_bn(m, g2[...], be2[...]), 0.0)
  hout_ref[...] = m
  onehot = (lax.broadcasted_iota(jnp.int32, (_G, _N), 0) ==
            b_ref[...]).astype(jnp.float32)
  y = y_ref[...] + jnp.dot(jnp.dot(onehot, m, precision=hp), wf[...],
                           precision=hp) + bf[...]
  if pool_input:
    y = y + jnp.dot(jnp.dot(onehot, h, precision=hp), wf0[...],
                    precision=hp) + bf0[...]
  yout_ref[...] = y


def _tc_layer(pool_input, h, agg, batch2d, p, fc, fc0, y):
  body = functools.partial(_tc_layer_body, pool_input)
  return pl.pallas_call(
      body,
      out_shape=(jax.ShapeDtypeStruct((_N, _H), jnp.float32),
                 jax.ShapeDtypeStruct((_G, _OUT), jnp.float32)),
  )(h, agg, batch2d,
    p['W1'], p['b1'].reshape(1, -1), p['g1'].reshape(1, -1),
    p['be1'].reshape(1, -1),
    p['W2'], p['b2'].reshape(1, -1), p['g'].reshape(1, -1),
    p['be'].reshape(1, -1),
    fc['W'], fc['b'].reshape(1, -1),
    fc0['W'], fc0['b'].reshape(1, -1), y)


def kernel(x, edge_index, batch, params):
  src = edge_index[0]
  dst = edge_index[1].reshape(_NW, _NWIN, _W)
  batch2d = batch.reshape(1, _N)
  y = jnp.zeros((_G, _OUT), jnp.float32)
  h = x
  for i in range(_L):
    agg = _sc_segment_sum(h, src, dst)
    h, y = _tc_layer(i == 0, h, agg, batch2d, params['conv%d' % i],
                     params['fcs'][i + 1], params['fcs'][0], y)
  return y


# speedup vs baseline: 11.0667x; 1.0093x over previous
import functools
import jax
import jax.numpy as jnp
from jax import lax
from jax.experimental import pallas as pl
from jax.experimental.pallas import tpu as pltpu
from jax.experimental.pallas import tpu_sc as plsc

_N = 10000
_E = 320000
_H = 128
_OUT = 64
_G = 128
_L = 4

_NC = 2
_NS = 16
_NW = _NC * _NS
_EPT = _E // _NW
_W = 80
_NWIN = _EPT // _W
_CPT = 10
_RPT = _N // _CPT


def _make_sc_segment_sum():
  mesh = plsc.VectorSubcoreMesh(core_axis_name="c", subcore_axis_name="s")

  @functools.partial(
      pl.kernel,
      out_type=jax.ShapeDtypeStruct((_NC, _N, _H), jnp.float32),
      mesh=mesh,
      scratch_types=[
          pltpu.VMEM((3, _W), jnp.int32),
          pltpu.VMEM((_NWIN, _W), jnp.int32),
          pltpu.VMEM((3, _W, _H), jnp.float32),
          pltpu.VMEM_SHARED((_N, _H), jnp.float32),
          pltpu.SemaphoreType.DMA,
          pltpu.SemaphoreType.DMA,
          pltpu.SemaphoreType.DMA,
          pltpu.SemaphoreType.DMA,
          pltpu.SemaphoreType.DMA,
          pltpu.SemaphoreType.DMA,
          pltpu.SemaphoreType.DMA,
      ],
  )
  def k(h_hbm, src_hbm, dst_hbm, out_hbm, sidx, didx, rows, acc,
        isem, gsem0, gsem1, gsem2, ssem0, ssem1, ssem2):
    core = lax.axis_index("c")
    sub = lax.axis_index("s")
    wid = sub * _NC + core
    ebase = wid * _EPT

    ic1 = pltpu.async_copy(dst_hbm.at[wid], didx, isem)

    @pl.when(sub < _CPT)
    def _():
      r0 = sub * _RPT
      pltpu.sync_copy(h_hbm.at[pl.ds(r0, _RPT)], acc.at[pl.ds(r0, _RPT)])
    ic1.wait()
    plsc.subcore_barrier()

    gsems = (gsem0, gsem1, gsem2)
    ssems = (ssem0, ssem1, ssem2)

    def sl_issue(w, b):
      pltpu.async_copy(src_hbm.at[pl.ds(ebase + w * _W, _W)], sidx.at[b],
                       ssems[b])

    def sl_wait(b):
      pltpu.make_async_copy(src_hbm.at[pl.ds(0, _W)], sidx.at[b],
                            ssems[b]).wait()

    def g_issue(b):
      pltpu.async_copy(h_hbm.at[sidx.at[b]], rows.at[b], gsems[b])

    def g_wait(b):
      pltpu.make_async_copy(h_hbm.at[sidx.at[0]], rows.at[b],
                            gsems[b]).wait()

    for b in range(3):
      sl_issue(b, b)
    for b in range(3):
      sl_wait(b)
      g_issue(b)

    @pl.loop(0, _NWIN - 2, step=3)
    def _(w):
      for b in range(3):
        ww = w + b
        g_wait(b)

        @pl.when(ww + 3 < _NWIN)
        def _():
          sl_issue(ww + 3, b)
        pltpu.sync_copy(rows.at[b], acc.at[didx.at[ww]], add=True)

        @pl.when(ww + 3 < _NWIN)
        def _():
          sl_wait(b)
          g_issue(b)

    for ww, b in ((_NWIN - 2, 0), (_NWIN - 1, 1)):
      g_wait(b)
      pltpu.sync_copy(rows.at[b], acc.at[didx.at[ww]], add=True)

    plsc.subcore_barrier()

    @pl.when(sub < _CPT)
    def _():
      r0 = sub * _RPT
      pltpu.sync_copy(acc.at[pl.ds(r0, _RPT)],
                      out_hbm.at[core, pl.ds(r0, _RPT)])

  return k


_sc_segment_sum = _make_sc_segment_sum()


def _bn(m, g, be):
  mu = jnp.mean(m, axis=0, keepdims=True)
  var = jnp.mean((m - mu) ** 2, axis=0, keepdims=True)
  return g * (m - mu) / jnp.sqrt(var + 1e-5) + be


_HP = jax.lax.Precision.HIGHEST


def _tc_layer_body(h_ref, a_ref, w1, bb1, g1, be1, w2, bb2, g2, be2,
                   hout_ref):
  z = a_ref[0] + a_ref[1] - h_ref[...]
  m = jnp.dot(z, w1[...], precision=_HP) + bb1[...]
  m = jnp.maximum(_bn(m, g1[...], be1[...]), 0.0)
  m = jnp.dot(m, w2[...], precision=_HP) + bb2[...]
  m = jnp.maximum(_bn(m, g2[...], be2[...]), 0.0)
  hout_ref[...] = m


def _tc_layer(h, agg, p):
  return pl.pallas_call(
      _tc_layer_body,
      out_shape=jax.ShapeDtypeStruct((_N, _H), jnp.float32),
  )(h, agg,
    p['W1'], p['b1'].reshape(1, -1), p['g1'].reshape(1, -1),
    p['be1'].reshape(1, -1),
    p['W2'], p['b2'].reshape(1, -1), p['g'].reshape(1, -1),
    p['be'].reshape(1, -1))


def _tc_pool_body(b_ref, h_ref, wf, bf, y_ref, yout_ref):
  onehot = (lax.broadcasted_iota(jnp.int32, (_G, _N), 0) ==
            b_ref[...]).astype(jnp.float32)
  pooled = jnp.dot(onehot, h_ref[...], precision=_HP)
  yout_ref[...] = y_ref[...] + jnp.dot(pooled, wf[...],
                                       precision=_HP) + bf[...]


def _tc_pool(batch2d, h, fc, y):
  return pl.pallas_call(
      _tc_pool_body,
      out_shape=jax.ShapeDtypeStruct((_G, _OUT), jnp.float32),
  )(batch2d, h, fc['W'], fc['b'].reshape(1, -1), y)


def kernel(x, edge_index, batch, params):
  src = edge_index[0]
  dst = edge_index[1].reshape(_NW, _NWIN, _W)
  batch2d = batch.reshape(1, _N)
  y = jnp.zeros((_G, _OUT), jnp.float32)
  h = x
  y = _tc_pool(batch2d, x, params['fcs'][0], y)
  for i in range(_L):
    agg = _sc_segment_sum(h, src, dst)
    h = _tc_layer(h, agg, params['conv%d' % i])
    y = _tc_pool(batch2d, h, params['fcs'][i + 1], y)
  return y
